# Initial kernel scaffold; baseline (speedup 1.0000x reference)
import jax
import jax.numpy as jnp
from jax.experimental import pallas as pl


def kernel(params, cell_x, piece_x, edge_index_occupies, edge_index_N, edge_index_E, cell_batch):
    raise NotImplementedError("write your pallas kernel here")


# speedup vs baseline: 1.0252x; 1.0252x over previous
import functools

import jax
import jax.numpy as jnp
from jax.experimental import pallas as pl
from jax.experimental.pallas import tpu as pltpu

_N_CELL = 10000
_N_PIECE = 5000
_D = 128
_H = 4
_C = 128
_NG = 64
_BW = 7


def _mm_body(x_ref, w_ref, b_ref, o_ref):
    o_ref[...] = (
        jnp.dot(x_ref[...], w_ref[...], preferred_element_type=jnp.float32)
        + b_ref[...]
    )


def _matmul(x, w, b, block_rows=1024):
    n, d = x.shape
    m = w.shape[1]
    npad = ((n + block_rows - 1) // block_rows) * block_rows
    if npad != n:
        x = jnp.pad(x, ((0, npad - n), (0, 0)))
    out = pl.pallas_call(
        _mm_body,
        grid=(npad // block_rows,),
        in_specs=[
            pl.BlockSpec((block_rows, d), lambda i: (i, 0)),
            pl.BlockSpec((d, m), lambda i: (0, 0)),
            pl.BlockSpec((1, m), lambda i: (0, 0)),
        ],
        out_specs=pl.BlockSpec((block_rows, m), lambda i: (i, 0)),
        out_shape=jax.ShapeDtypeStruct((npad, m), jnp.float32),
    )(x, w, b[None])
    return out[:n]


def _gat(x_src, x_dst, ei, p):
    xl = _matmul(x_src, p["Wl"], p["bl"]).reshape(-1, _H, _C)
    xr = _matmul(x_dst, p["Wr"], p["br"]).reshape(-1, _H, _C)
    src = ei[0]
    dst = ei[1]
    n_dst = x_dst.shape[0]
    z = jax.nn.leaky_relu(xl[src] + xr[dst], 0.2)
    e = jnp.sum(z * p["att"][None, :, :], axis=-1)
    m = jax.ops.segment_max(e, dst, num_segments=n_dst)
    m = jnp.where(jnp.isfinite(m), m, 0.0)
    ex = jnp.exp(e - m[dst])
    den = jax.ops.segment_sum(ex, dst, num_segments=n_dst)
    alpha = ex / (den[dst] + 1e-16)
    out = jax.ops.segment_sum(xl[src] * alpha[..., None], dst, num_segments=n_dst)
    return out.mean(axis=1) + p["b"]


def kernel(params, cell_x, piece_x, edge_index_occupies, edge_index_N, edge_index_E, cell_batch):
    cell = params["cell_emb"][cell_x]
    piece = params["piece_emb"][piece_x[:, 0]]
    for lp in params["layers"]:
        new_cell = (
            _gat(piece, cell, edge_index_occupies, lp["occ"])
            + _gat(cell, cell, edge_index_N, lp["N"])
            + _gat(cell, cell, edge_index_E, lp["E"])
        )
        cell = jax.nn.relu(new_cell)
        piece = jax.nn.relu(piece)
    sums = jax.ops.segment_sum(cell, cell_batch, num_segments=_NG)
    cnt = jax.ops.segment_sum(
        jnp.ones((cell.shape[0],), cell.dtype), cell_batch, num_segments=_NG
    )
    graph_emb = sums / jnp.maximum(cnt, 1.0)[:, None]
    h = jax.nn.relu(_matmul(graph_emb, params["fc1_W"], params["fc1_b"], block_rows=64))
    policy_logits = _matmul(h, params["pol_W"], params["pol_b"], block_rows=64)
    value = jnp.tanh(_matmul(h, params["val_W"], params["val_b"], block_rows=64))
    return policy_logits, value


# speedup vs baseline: 3.9010x; 3.8049x over previous
import functools

import jax
import jax.numpy as jnp
from jax import lax
from jax.experimental import pallas as pl
from jax.experimental.pallas import tpu as pltpu
from jax.experimental.pallas import tpu_sc as plsc

_N_CELL = 10000
_N_PIECE = 5000
_D = 128
_H = 4
_C = 128
_HC = _H * _C
_NG = 64

_NWORKERS = 32
_NDST_PAD = 10240
_QD = _NDST_PAD // _NWORKERS


def _mm_body(x_ref, w_ref, b_ref, o_ref):
    o_ref[...] = (
        jnp.dot(x_ref[...], w_ref[...], preferred_element_type=jnp.float32)
        + b_ref[...]
    )


def _matmul(x, w, b, block_rows=1024):
    n, d = x.shape
    m = w.shape[1]
    npad = ((n + block_rows - 1) // block_rows) * block_rows
    if npad != n:
        x = jnp.pad(x, ((0, npad - n), (0, 0)))
    out = pl.pallas_call(
        _mm_body,
        grid=(npad // block_rows,),
        in_specs=[
            pl.BlockSpec((block_rows, d), lambda i: (i, 0)),
            pl.BlockSpec((d, m), lambda i: (0, 0)),
            pl.BlockSpec((1, m), lambda i: (0, 0)),
        ],
        out_specs=pl.BlockSpec((block_rows, m), lambda i: (i, 0)),
        out_shape=jax.ShapeDtypeStruct((npad, m), jnp.float32),
    )(x, w, b[None])
    return out[:n]


def _dyn_take(vec, idx16):
    return lax.gather(
        vec, idx16[:, None],
        lax.GatherDimensionNumbers(offset_dims=(), collapsed_slice_dims=(0,),
                                   start_index_map=(0,)),
        (1,), mode=lax.GatherScatterMode.PROMISE_IN_BOUNDS)

def _sc_gat_body(xl_hbm, xr_hbm, srcs_hbm, ptr_hbm, deg_hbm, att_hbm, out_hbm,
                 ptr_v, deg_v, att_v, xr_v, idx_v, rows_v, acc_v,
                 outrow_v, sem):
    nc = 2
    wid = lax.axis_index("s") * nc + lax.axis_index("c")
    base = wid * _QD

    pltpu.sync_copy(ptr_hbm.at[pl.ds(pl.multiple_of(base, _QD), _QD)], ptr_v)
    pltpu.sync_copy(deg_hbm.at[pl.ds(pl.multiple_of(base, _QD), _QD)], deg_v)
    pltpu.sync_copy(att_hbm, att_v)

    neg_big = jnp.float32(-1e30)

    def dst_body(d, _):
        g16 = pl.multiple_of((d // 16) * 16, 16)
        lane = jnp.full((16,), d - g16, jnp.int32)
        dvec = deg_v[pl.ds(g16, 16)]
        pvec = ptr_v[pl.ds(g16, 16)]
        deg = _dyn_take(dvec, lane)[0]
        start = _dyn_take(pvec, lane)[0]
        pltpu.sync_copy(xr_hbm.at[base + d], xr_v)
        for h in range(_H):
            for b in range(8):
                acc_v[h, pl.ds(16 * b, 16)] = jnp.zeros((16,), jnp.float32)

        nch = (deg + 15) // 16

        def chunk_body(c, carry):
            m0, m1, m2, m3, d0, d1, d2, d3 = carry
            ms = [m0, m1, m2, m3]
            dens = [d0, d1, d2, d3]
            off = pl.multiple_of(start + c * 16, 16)
            pltpu.sync_copy(srcs_hbm.at[pl.ds(off, 16)], idx_v)
            pltpu.async_copy(xl_hbm.at[idx_v], rows_v, sem).wait()
            rem = jnp.minimum(deg - c * 16, 16)

            e_init = tuple(jnp.full((16,), neg_big) for _ in range(_H))
            lanes = lax.iota(jnp.int32, 16)

            def edge_body(j, evecs):
                accs = [jnp.zeros((16,), jnp.float32) for _ in range(_H)]
                for b in range(32):
                    h = b // 8
                    r = rows_v[j, pl.ds(16 * b, 16)]
                    t = r + xr_v[pl.ds(16 * b, 16)]
                    z = 0.6 * t + 0.4 * jnp.abs(t)
                    accs[h] = accs[h] + z * att_v[pl.ds(16 * b, 16)]
                out = []
                for h in range(_H):
                    s = jnp.sum(accs[h])
                    out.append(jnp.where(lanes == j, s, evecs[h]))
                return tuple(out)

            evecs = lax.fori_loop(0, rem, edge_body, e_init)

            new_ms = []
            new_dens = []
            p_list = []
            for h in range(_H):
                cmax = jnp.max(evecs[h])
                new_m = jnp.maximum(ms[h], cmax)
                sv = jnp.exp(jnp.full((16,), ms[h] - new_m))
                p_h = jnp.exp(evecs[h] - new_m)
                p_list.append(p_h)
                new_dens.append(dens[h] * sv + jnp.sum(p_h))
                for b in range(8):
                    acc_v[h, pl.ds(16 * b, 16)] = (
                        acc_v[h, pl.ds(16 * b, 16)] * sv
                    )
                new_ms.append(new_m)

            def w_body(j, _):
                jsplat = jnp.full((16,), j, jnp.int32)
                for h in range(_H):
                    pj = _dyn_take(p_list[h], jsplat)
                    for b in range(8):
                        acc_v[h, pl.ds(16 * b, 16)] = (
                            acc_v[h, pl.ds(16 * b, 16)]
                            + pj * rows_v[j, pl.ds(h * 128 + 16 * b, 16)]
                        )
                return 0

            lax.fori_loop(0, rem, w_body, 0)
            return (new_ms[0], new_ms[1], new_ms[2], new_ms[3],
                    new_dens[0], new_dens[1], new_dens[2], new_dens[3])

        init = (neg_big, neg_big, neg_big, neg_big,
                jnp.zeros((16,), jnp.float32), jnp.zeros((16,), jnp.float32),
                jnp.zeros((16,), jnp.float32), jnp.zeros((16,), jnp.float32))
        carry = lax.fori_loop(0, nch, chunk_body, init)
        dens = carry[4:]

        invs = [0.25 / (dens[h] + 1e-16) for h in range(_H)]
        for b in range(8):
            o = jnp.zeros((16,), jnp.float32)
            for h in range(_H):
                o = o + acc_v[h, pl.ds(16 * b, 16)] * invs[h]
            outrow_v[pl.ds(16 * b, 16)] = o
        pltpu.sync_copy(outrow_v, out_hbm.at[base + d])
        return 0

    lax.fori_loop(0, _QD, dst_body, 0)


@functools.cache
def _sc_gat_kernel(n_src, p_max):
    mesh = plsc.VectorSubcoreMesh(core_axis_name="c", subcore_axis_name="s")
    return functools.partial(
        pl.kernel,
        mesh=mesh,
        compiler_params=pltpu.CompilerParams(needs_layout_passes=False),
        out_type=jax.ShapeDtypeStruct((_NDST_PAD, _C), jnp.float32),
        scratch_types=[
            pltpu.VMEM((_QD,), jnp.int32),
            pltpu.VMEM((_QD,), jnp.int32),
            pltpu.VMEM((_HC,), jnp.float32),
            pltpu.VMEM((_HC,), jnp.float32),
            pltpu.VMEM((16,), jnp.int32),
            pltpu.VMEM((16, _HC), jnp.float32),
            pltpu.VMEM((_H, _C), jnp.float32),
            pltpu.VMEM((_C,), jnp.float32),
            pltpu.SemaphoreType.DMA,
        ],
    )(_sc_gat_body)


def _build_csr(dst, n_edges, p_max):
    ones = jnp.ones((n_edges,), jnp.int32)
    deg = jax.ops.segment_sum(ones, dst, num_segments=_NDST_PAD)
    cap = ((deg + 15) // 16) * 16
    ptr = jnp.concatenate([jnp.zeros((1,), jnp.int32),
                           jnp.cumsum(cap)[:-1].astype(jnp.int32)])
    excl = jnp.concatenate([jnp.zeros((1,), jnp.int32),
                            jnp.cumsum(deg)[:-1].astype(jnp.int32)])
    order = jnp.argsort(dst)
    pos = ptr[dst[order]] + (jnp.arange(n_edges, dtype=jnp.int32)
                             - excl[dst[order]])
    return ptr, deg.astype(jnp.int32), pos, order


def _gat_sc(x_src, xr_pad, ei, p, csr):
    ptr, deg, srcs_padded = csr
    n_src = x_src.shape[0]
    xl = _matmul(x_src, p["Wl"], p["bl"])
    att = p["att"].reshape(_HC)
    fn = _sc_gat_kernel(n_src, srcs_padded.shape[0])
    out = fn(xl, xr_pad, srcs_padded, ptr, deg, att)
    return out


def kernel(params, cell_x, piece_x, edge_index_occupies, edge_index_N, edge_index_E, cell_batch):
    cell = params["cell_emb"][cell_x]
    piece = params["piece_emb"][piece_x[:, 0]]

    csrs = []
    for ei, n_edges in ((edge_index_occupies, 20000),
                        (edge_index_N, 320000),
                        (edge_index_E, 320000)):
        p_max = n_edges + 16 * _NDST_PAD
        src, dst = ei[0], ei[1]
        ptr, deg, pos, order = _build_csr(dst, n_edges, p_max)
        srcs_padded = jnp.zeros((p_max,), jnp.int32).at[pos].set(src[order])
        csrs.append((ptr, deg, srcs_padded))
    csr_occ, csr_n, csr_e = csrs

    for lp in params["layers"]:
        cell_pad = jnp.pad(cell, ((0, _NDST_PAD - _N_CELL), (0, 0)))
        s_occ = _gat_sc(piece, _matmul(cell_pad, lp["occ"]["Wr"], lp["occ"]["br"]),
                        edge_index_occupies, lp["occ"], csr_occ)
        s_n = _gat_sc(cell, _matmul(cell_pad, lp["N"]["Wr"], lp["N"]["br"]),
                      edge_index_N, lp["N"], csr_n)
        s_e = _gat_sc(cell, _matmul(cell_pad, lp["E"]["Wr"], lp["E"]["br"]),
                      edge_index_E, lp["E"], csr_e)
        bias = lp["occ"]["b"] + lp["N"]["b"] + lp["E"]["b"]
        cell = jax.nn.relu((s_occ + s_n + s_e)[:_N_CELL] + bias)
        piece = jax.nn.relu(piece)

    sums = jax.ops.segment_sum(cell, cell_batch, num_segments=_NG)
    cnt = jax.ops.segment_sum(
        jnp.ones((cell.shape[0],), cell.dtype), cell_batch, num_segments=_NG
    )
    graph_emb = sums / jnp.maximum(cnt, 1.0)[:, None]
    h = jax.nn.relu(_matmul(graph_emb, params["fc1_W"], params["fc1_b"], block_rows=64))
    policy_logits = _matmul(h, params["pol_W"], params["pol_b"], block_rows=64)
    value = jnp.tanh(_matmul(h, params["val_W"], params["val_b"], block_rows=64))
    return policy_logits, value


# speedup vs baseline: 4.2335x; 1.0852x over previous
import functools

import jax
import jax.numpy as jnp
from jax import lax
from jax.experimental import pallas as pl
from jax.experimental.pallas import tpu as pltpu
from jax.experimental.pallas import tpu_sc as plsc

_N_CELL = 10000
_N_PIECE = 5000
_D = 128
_H = 4
_C = 128
_HC = _H * _C
_NG = 64

_NWORKERS = 32
_NDST_PAD = 10240
_QD = _NDST_PAD // _NWORKERS


def _mm_body(x_ref, w_ref, b_ref, o_ref):
    o_ref[...] = (
        jnp.dot(x_ref[...], w_ref[...], preferred_element_type=jnp.float32)
        + b_ref[...]
    )


def _matmul(x, w, b, block_rows=1024):
    n, d = x.shape
    m = w.shape[1]
    npad = ((n + block_rows - 1) // block_rows) * block_rows
    if npad != n:
        x = jnp.pad(x, ((0, npad - n), (0, 0)))
    out = pl.pallas_call(
        _mm_body,
        grid=(npad // block_rows,),
        in_specs=[
            pl.BlockSpec((block_rows, d), lambda i: (i, 0)),
            pl.BlockSpec((d, m), lambda i: (0, 0)),
            pl.BlockSpec((1, m), lambda i: (0, 0)),
        ],
        out_specs=pl.BlockSpec((block_rows, m), lambda i: (i, 0)),
        out_shape=jax.ShapeDtypeStruct((npad, m), jnp.float32),
    )(x, w, b[None])
    return out[:n]


def _dyn_take(vec, idx16):
    return lax.gather(
        vec, idx16[:, None],
        lax.GatherDimensionNumbers(offset_dims=(), collapsed_slice_dims=(0,),
                                   start_index_map=(0,)),
        (1,), mode=lax.GatherScatterMode.PROMISE_IN_BOUNDS)

def _sc_gat_body(xl_hbm, xr_hbm, srcs_hbm, ptr_hbm, deg_hbm, att_hbm, out_hbm,
                 ptr_v, deg_v, att_v, xr_v, idx_a, idx_b, rows_a, rows_b,
                 acc_v, outrow_v, sem_a, sem_b):
    nc = 2
    wid = lax.axis_index("s") * nc + lax.axis_index("c")
    base = wid * _QD

    pltpu.sync_copy(ptr_hbm.at[pl.ds(pl.multiple_of(base, _QD), _QD)], ptr_v)
    pltpu.sync_copy(deg_hbm.at[pl.ds(pl.multiple_of(base, _QD), _QD)], deg_v)
    pltpu.sync_copy(att_hbm, att_v)

    neg_big = jnp.float32(-1e30)

    def dst_body(d, _):
        g16 = pl.multiple_of((d // 16) * 16, 16)
        lane = jnp.full((16,), d - g16, jnp.int32)
        dvec = deg_v[pl.ds(g16, 16)]
        pvec = ptr_v[pl.ds(g16, 16)]
        deg = _dyn_take(dvec, lane)[0]
        start = _dyn_take(pvec, lane)[0]
        pltpu.sync_copy(xr_hbm.at[base + d], xr_v)
        for h in range(_H):
            for b in range(8):
                acc_v[h, pl.ds(16 * b, 16)] = jnp.zeros((16,), jnp.float32)

        nch = (deg + 15) // 16

        def process(rows_v, carry, rem):
            ms = list(carry[0:4])
            dens = list(carry[4:8])

            e_init = tuple(jnp.full((16,), neg_big) for _ in range(_H))
            lanes = lax.iota(jnp.int32, 16)

            def edge_body(j, evecs):
                accs = [jnp.zeros((16,), jnp.float32) for _ in range(_H)]
                for b in range(32):
                    h = b // 8
                    r = rows_v[j, pl.ds(16 * b, 16)]
                    t = r + xr_v[pl.ds(16 * b, 16)]
                    z = 0.6 * t + 0.4 * jnp.abs(t)
                    accs[h] = accs[h] + z * att_v[pl.ds(16 * b, 16)]
                out = []
                for h in range(_H):
                    s = jnp.sum(accs[h])
                    out.append(jnp.where(lanes == j, s, evecs[h]))
                return tuple(out)

            evecs = lax.fori_loop(0, rem, edge_body, e_init)

            new_ms = []
            new_dens = []
            p_list = []
            for h in range(_H):
                cmax = jnp.max(evecs[h])
                new_m = jnp.maximum(ms[h], cmax)
                sv = jnp.exp(jnp.full((16,), ms[h] - new_m))
                p_h = jnp.exp(evecs[h] - new_m)
                p_list.append(p_h)
                new_dens.append(dens[h] * sv + jnp.sum(p_h))
                for b in range(8):
                    acc_v[h, pl.ds(16 * b, 16)] = (
                        acc_v[h, pl.ds(16 * b, 16)] * sv
                    )
                new_ms.append(new_m)

            def w_body(j, _):
                jsplat = jnp.full((16,), j, jnp.int32)
                for h in range(_H):
                    pj = _dyn_take(p_list[h], jsplat)
                    for b in range(8):
                        acc_v[h, pl.ds(16 * b, 16)] = (
                            acc_v[h, pl.ds(16 * b, 16)]
                            + pj * rows_v[j, pl.ds(h * 128 + 16 * b, 16)]
                        )
                return 0

            lax.fori_loop(0, rem, w_body, 0)
            return (new_ms[0], new_ms[1], new_ms[2], new_ms[3],
                    new_dens[0], new_dens[1], new_dens[2], new_dens[3])

        @pl.when(nch > 0)
        def _prologue():
            off0 = pl.multiple_of(start, 16)
            pltpu.sync_copy(srcs_hbm.at[pl.ds(off0, 16)], idx_a)
            pltpu.async_copy(xl_hbm.at[idx_a], rows_a, sem_a)

        def chunk_body(c, carry):
            even = (c % 2) == 0

            @pl.when(c + 1 < nch)
            def _prefetch():
                offn = pl.multiple_of(start + (c + 1) * 16, 16)

                @pl.when(even)
                def _():
                    pltpu.sync_copy(srcs_hbm.at[pl.ds(offn, 16)], idx_b)
                    pltpu.async_copy(xl_hbm.at[idx_b], rows_b, sem_b)

                @pl.when(jnp.logical_not(even))
                def _():
                    pltpu.sync_copy(srcs_hbm.at[pl.ds(offn, 16)], idx_a)
                    pltpu.async_copy(xl_hbm.at[idx_a], rows_a, sem_a)

            rem = jnp.minimum(deg - c * 16, 16)

            def on_a(cr):
                pltpu.make_async_copy(xl_hbm.at[idx_a], rows_a, sem_a).wait()
                return process(rows_a, cr, rem)

            def on_b(cr):
                pltpu.make_async_copy(xl_hbm.at[idx_b], rows_b, sem_b).wait()
                return process(rows_b, cr, rem)

            return lax.cond(even, on_a, on_b, carry)

        init = (neg_big, neg_big, neg_big, neg_big,
                jnp.zeros((16,), jnp.float32), jnp.zeros((16,), jnp.float32),
                jnp.zeros((16,), jnp.float32), jnp.zeros((16,), jnp.float32))
        carry = lax.fori_loop(0, nch, chunk_body, init)
        dens = carry[4:]

        invs = [0.25 / (dens[h] + 1e-16) for h in range(_H)]
        for b in range(8):
            o = jnp.zeros((16,), jnp.float32)
            for h in range(_H):
                o = o + acc_v[h, pl.ds(16 * b, 16)] * invs[h]
            outrow_v[pl.ds(16 * b, 16)] = o
        pltpu.sync_copy(outrow_v, out_hbm.at[base + d])
        return 0

    lax.fori_loop(0, _QD, dst_body, 0)


@functools.cache
def _sc_gat_kernel(n_src, p_max):
    mesh = plsc.VectorSubcoreMesh(core_axis_name="c", subcore_axis_name="s")
    return functools.partial(
        pl.kernel,
        mesh=mesh,
        compiler_params=pltpu.CompilerParams(needs_layout_passes=False),
        out_type=jax.ShapeDtypeStruct((_NDST_PAD, _C), jnp.float32),
        scratch_types=[
            pltpu.VMEM((_QD,), jnp.int32),
            pltpu.VMEM((_QD,), jnp.int32),
            pltpu.VMEM((_HC,), jnp.float32),
            pltpu.VMEM((_HC,), jnp.float32),
            pltpu.VMEM((16,), jnp.int32),
            pltpu.VMEM((16,), jnp.int32),
            pltpu.VMEM((16, _HC), jnp.float32),
            pltpu.VMEM((16, _HC), jnp.float32),
            pltpu.VMEM((_H, _C), jnp.float32),
            pltpu.VMEM((_C,), jnp.float32),
            pltpu.SemaphoreType.DMA,
            pltpu.SemaphoreType.DMA,
        ],
    )(_sc_gat_body)


def _build_csr(dst, n_edges, p_max):
    ones = jnp.ones((n_edges,), jnp.int32)
    deg = jax.ops.segment_sum(ones, dst, num_segments=_NDST_PAD)
    cap = ((deg + 15) // 16) * 16
    ptr = jnp.concatenate([jnp.zeros((1,), jnp.int32),
                           jnp.cumsum(cap)[:-1].astype(jnp.int32)])
    excl = jnp.concatenate([jnp.zeros((1,), jnp.int32),
                            jnp.cumsum(deg)[:-1].astype(jnp.int32)])
    order = jnp.argsort(dst)
    pos = ptr[dst[order]] + (jnp.arange(n_edges, dtype=jnp.int32)
                             - excl[dst[order]])
    return ptr, deg.astype(jnp.int32), pos, order


def _gat_sc(x_src, xr_pad, ei, p, csr):
    ptr, deg, srcs_padded = csr
    n_src = x_src.shape[0]
    xl = _matmul(x_src, p["Wl"], p["bl"])
    att = p["att"].reshape(_HC)
    fn = _sc_gat_kernel(n_src, srcs_padded.shape[0])
    out = fn(xl, xr_pad, srcs_padded, ptr, deg, att)
    return out


def kernel(params, cell_x, piece_x, edge_index_occupies, edge_index_N, edge_index_E, cell_batch):
    cell = params["cell_emb"][cell_x]
    piece = params["piece_emb"][piece_x[:, 0]]

    csrs = []
    for ei, n_edges in ((edge_index_occupies, 20000),
                        (edge_index_N, 320000),
                        (edge_index_E, 320000)):
        p_max = n_edges + 16 * _NDST_PAD
        src, dst = ei[0], ei[1]
        ptr, deg, pos, order = _build_csr(dst, n_edges, p_max)
        srcs_padded = jnp.zeros((p_max,), jnp.int32).at[pos].set(src[order])
        csrs.append((ptr, deg, srcs_padded))
    csr_occ, csr_n, csr_e = csrs

    for lp in params["layers"]:
        cell_pad = jnp.pad(cell, ((0, _NDST_PAD - _N_CELL), (0, 0)))
        s_occ = _gat_sc(piece, _matmul(cell_pad, lp["occ"]["Wr"], lp["occ"]["br"]),
                        edge_index_occupies, lp["occ"], csr_occ)
        s_n = _gat_sc(cell, _matmul(cell_pad, lp["N"]["Wr"], lp["N"]["br"]),
                      edge_index_N, lp["N"], csr_n)
        s_e = _gat_sc(cell, _matmul(cell_pad, lp["E"]["Wr"], lp["E"]["br"]),
                      edge_index_E, lp["E"], csr_e)
        bias = lp["occ"]["b"] + lp["N"]["b"] + lp["E"]["b"]
        cell = jax.nn.relu((s_occ + s_n + s_e)[:_N_CELL] + bias)
        piece = jax.nn.relu(piece)

    sums = jax.ops.segment_sum(cell, cell_batch, num_segments=_NG)
    cnt = jax.ops.segment_sum(
        jnp.ones((cell.shape[0],), cell.dtype), cell_batch, num_segments=_NG
    )
    graph_emb = sums / jnp.maximum(cnt, 1.0)[:, None]
    h = jax.nn.relu(_matmul(graph_emb, params["fc1_W"], params["fc1_b"], block_rows=64))
    policy_logits = _matmul(h, params["pol_W"], params["pol_b"], block_rows=64)
    value = jnp.tanh(_matmul(h, params["val_W"], params["val_b"], block_rows=64))
    return policy_logits, value


# speedup vs baseline: 4.2780x; 1.0105x over previous
import functools

import jax
import jax.numpy as jnp
from jax import lax
from jax.experimental import pallas as pl
from jax.experimental.pallas import tpu as pltpu
from jax.experimental.pallas import tpu_sc as plsc

_N_CELL = 10000
_N_PIECE = 5000
_D = 128
_H = 4
_C = 128
_HC = _H * _C
_NG = 64

_NWORKERS = 32
_NDST_PAD = 10240
_QD = _NDST_PAD // _NWORKERS


def _mm_body(x_ref, w_ref, b_ref, o_ref):
    o_ref[...] = (
        jnp.dot(x_ref[...], w_ref[...], preferred_element_type=jnp.float32)
        + b_ref[...]
    )


def _matmul(x, w, b, block_rows=1024):
    n, d = x.shape
    m = w.shape[1]
    npad = ((n + block_rows - 1) // block_rows) * block_rows
    if npad != n:
        x = jnp.pad(x, ((0, npad - n), (0, 0)))
    out = pl.pallas_call(
        _mm_body,
        grid=(npad // block_rows,),
        in_specs=[
            pl.BlockSpec((block_rows, d), lambda i: (i, 0)),
            pl.BlockSpec((d, m), lambda i: (0, 0)),
            pl.BlockSpec((1, m), lambda i: (0, 0)),
        ],
        out_specs=pl.BlockSpec((block_rows, m), lambda i: (i, 0)),
        out_shape=jax.ShapeDtypeStruct((npad, m), jnp.float32),
    )(x, w, b[None])
    return out[:n]


def _dyn_take(vec, idx16):
    return lax.gather(
        vec, idx16[:, None],
        lax.GatherDimensionNumbers(offset_dims=(), collapsed_slice_dims=(0,),
                                   start_index_map=(0,)),
        (1,), mode=lax.GatherScatterMode.PROMISE_IN_BOUNDS)

def _sc_gat_body(xl_hbm, xr_hbm, srcs_hbm, ptr_hbm, deg_hbm, att_hbm, out_hbm,
                 ptr_v, deg_v, att_v, xr_v, idx_a, idx_b, rows_a, rows_b,
                 acc_v, outrow_v, sem_a, sem_b):
    nc = 2
    wid = lax.axis_index("s") * nc + lax.axis_index("c")
    base = wid * _QD

    pltpu.sync_copy(ptr_hbm.at[pl.ds(pl.multiple_of(base, _QD), _QD)], ptr_v)
    pltpu.sync_copy(deg_hbm.at[pl.ds(pl.multiple_of(base, _QD), _QD)], deg_v)
    pltpu.sync_copy(att_hbm, att_v)

    neg_big = jnp.float32(-1e30)

    def group_body(g, _):
        g16 = pl.multiple_of(g * 16, 16)
        dvec = deg_v[pl.ds(g16, 16)]
        pvec = ptr_v[pl.ds(g16, 16)]
        pltpu.sync_copy(xr_hbm.at[pl.ds(base + g16, 16)], xr_v)

        lax.fori_loop(0, 16, functools.partial(dst_body, dvec, pvec), 0)
        pltpu.sync_copy(outrow_v, out_hbm.at[pl.ds(base + g16, 16)])
        return 0

    def dst_body(dvec, pvec, jj, _):
        lane = jnp.full((16,), jj, jnp.int32)
        deg = _dyn_take(dvec, lane)[0]
        start = _dyn_take(pvec, lane)[0]
        for h in range(_H):
            for b in range(8):
                acc_v[h, pl.ds(16 * b, 16)] = jnp.zeros((16,), jnp.float32)

        nch = (deg + 15) // 16

        def process(rows_v, carry, rem):
            ms = list(carry[0:4])
            dens = list(carry[4:8])

            e_init = tuple(jnp.full((16,), neg_big) for _ in range(_H))
            lanes = lax.iota(jnp.int32, 16)

            def edge_body(j, evecs):
                accs = [jnp.zeros((16,), jnp.float32) for _ in range(_H)]
                for b in range(32):
                    h = b // 8
                    r = rows_v[j, pl.ds(16 * b, 16)]
                    t = r + xr_v[jj, pl.ds(16 * b, 16)]
                    z = 0.6 * t + 0.4 * jnp.abs(t)
                    accs[h] = accs[h] + z * att_v[pl.ds(16 * b, 16)]
                out = []
                for h in range(_H):
                    s = jnp.sum(accs[h])
                    out.append(jnp.where(lanes == j, s, evecs[h]))
                return tuple(out)

            evecs = lax.fori_loop(0, rem, edge_body, e_init)

            new_ms = []
            new_dens = []
            p_list = []
            for h in range(_H):
                cmax = jnp.max(evecs[h])
                new_m = jnp.maximum(ms[h], cmax)
                sv = jnp.exp(jnp.full((16,), ms[h] - new_m))
                p_h = jnp.exp(evecs[h] - new_m)
                p_list.append(p_h)
                new_dens.append(dens[h] * sv + jnp.sum(p_h))
                for b in range(8):
                    acc_v[h, pl.ds(16 * b, 16)] = (
                        acc_v[h, pl.ds(16 * b, 16)] * sv
                    )
                new_ms.append(new_m)

            def w_body(j, _):
                jsplat = jnp.full((16,), j, jnp.int32)
                for h in range(_H):
                    pj = _dyn_take(p_list[h], jsplat)
                    for b in range(8):
                        acc_v[h, pl.ds(16 * b, 16)] = (
                            acc_v[h, pl.ds(16 * b, 16)]
                            + pj * rows_v[j, pl.ds(h * 128 + 16 * b, 16)]
                        )
                return 0

            lax.fori_loop(0, rem, w_body, 0)
            return (new_ms[0], new_ms[1], new_ms[2], new_ms[3],
                    new_dens[0], new_dens[1], new_dens[2], new_dens[3])

        @pl.when(nch > 0)
        def _prologue():
            off0 = pl.multiple_of(start, 16)
            pltpu.sync_copy(srcs_hbm.at[pl.ds(off0, 16)], idx_a)
            pltpu.async_copy(xl_hbm.at[idx_a], rows_a, sem_a)

        def chunk_body(c, carry):
            even = (c % 2) == 0

            @pl.when(c + 1 < nch)
            def _prefetch():
                offn = pl.multiple_of(start + (c + 1) * 16, 16)

                @pl.when(even)
                def _():
                    pltpu.sync_copy(srcs_hbm.at[pl.ds(offn, 16)], idx_b)
                    pltpu.async_copy(xl_hbm.at[idx_b], rows_b, sem_b)

                @pl.when(jnp.logical_not(even))
                def _():
                    pltpu.sync_copy(srcs_hbm.at[pl.ds(offn, 16)], idx_a)
                    pltpu.async_copy(xl_hbm.at[idx_a], rows_a, sem_a)

            rem = jnp.minimum(deg - c * 16, 16)

            def on_a(cr):
                pltpu.make_async_copy(xl_hbm.at[idx_a], rows_a, sem_a).wait()
                return process(rows_a, cr, rem)

            def on_b(cr):
                pltpu.make_async_copy(xl_hbm.at[idx_b], rows_b, sem_b).wait()
                return process(rows_b, cr, rem)

            return lax.cond(even, on_a, on_b, carry)

        init = (neg_big, neg_big, neg_big, neg_big,
                jnp.zeros((16,), jnp.float32), jnp.zeros((16,), jnp.float32),
                jnp.zeros((16,), jnp.float32), jnp.zeros((16,), jnp.float32))
        carry = lax.fori_loop(0, nch, chunk_body, init)
        dens = carry[4:]

        invs = [0.25 / (dens[h] + 1e-16) for h in range(_H)]
        for b in range(8):
            o = jnp.zeros((16,), jnp.float32)
            for h in range(_H):
                o = o + acc_v[h, pl.ds(16 * b, 16)] * invs[h]
            outrow_v[jj, pl.ds(16 * b, 16)] = o
        return 0

    lax.fori_loop(0, _QD // 16, group_body, 0)


@functools.cache
def _sc_gat_kernel(n_src, p_max):
    mesh = plsc.VectorSubcoreMesh(core_axis_name="c", subcore_axis_name="s")
    return functools.partial(
        pl.kernel,
        mesh=mesh,
        compiler_params=pltpu.CompilerParams(needs_layout_passes=False),
        out_type=jax.ShapeDtypeStruct((_NDST_PAD, _C), jnp.float32),
        scratch_types=[
            pltpu.VMEM((_QD,), jnp.int32),
            pltpu.VMEM((_QD,), jnp.int32),
            pltpu.VMEM((_HC,), jnp.float32),
            pltpu.VMEM((16, _HC), jnp.float32),
            pltpu.VMEM((16,), jnp.int32),
            pltpu.VMEM((16,), jnp.int32),
            pltpu.VMEM((16, _HC), jnp.float32),
            pltpu.VMEM((16, _HC), jnp.float32),
            pltpu.VMEM((_H, _C), jnp.float32),
            pltpu.VMEM((16, _C), jnp.float32),
            pltpu.SemaphoreType.DMA,
            pltpu.SemaphoreType.DMA,
        ],
    )(_sc_gat_body)


def _build_csr(dst, n_edges, p_max):
    ones = jnp.ones((n_edges,), jnp.int32)
    deg = jax.ops.segment_sum(ones, dst, num_segments=_NDST_PAD)
    cap = ((deg + 15) // 16) * 16
    ptr = jnp.concatenate([jnp.zeros((1,), jnp.int32),
                           jnp.cumsum(cap)[:-1].astype(jnp.int32)])
    excl = jnp.concatenate([jnp.zeros((1,), jnp.int32),
                            jnp.cumsum(deg)[:-1].astype(jnp.int32)])
    order = jnp.argsort(dst)
    pos = ptr[dst[order]] + (jnp.arange(n_edges, dtype=jnp.int32)
                             - excl[dst[order]])
    return ptr, deg.astype(jnp.int32), pos, order


def _gat_sc(x_src, xr_pad, ei, p, csr):
    ptr, deg, srcs_padded = csr
    n_src = x_src.shape[0]
    xl = _matmul(x_src, p["Wl"], p["bl"])
    att = p["att"].reshape(_HC)
    fn = _sc_gat_kernel(n_src, srcs_padded.shape[0])
    out = fn(xl, xr_pad, srcs_padded, ptr, deg, att)
    return out


def kernel(params, cell_x, piece_x, edge_index_occupies, edge_index_N, edge_index_E, cell_batch):
    cell = params["cell_emb"][cell_x]
    piece = params["piece_emb"][piece_x[:, 0]]

    csrs = []
    for ei, n_edges in ((edge_index_occupies, 20000),
                        (edge_index_N, 320000),
                        (edge_index_E, 320000)):
        p_max = n_edges + 16 * _NDST_PAD
        src, dst = ei[0], ei[1]
        ptr, deg, pos, order = _build_csr(dst, n_edges, p_max)
        srcs_padded = jnp.zeros((p_max,), jnp.int32).at[pos].set(src[order])
        csrs.append((ptr, deg, srcs_padded))
    csr_occ, csr_n, csr_e = csrs

    for lp in params["layers"]:
        cell_pad = jnp.pad(cell, ((0, _NDST_PAD - _N_CELL), (0, 0)))
        s_occ = _gat_sc(piece, _matmul(cell_pad, lp["occ"]["Wr"], lp["occ"]["br"]),
                        edge_index_occupies, lp["occ"], csr_occ)
        s_n = _gat_sc(cell, _matmul(cell_pad, lp["N"]["Wr"], lp["N"]["br"]),
                      edge_index_N, lp["N"], csr_n)
        s_e = _gat_sc(cell, _matmul(cell_pad, lp["E"]["Wr"], lp["E"]["br"]),
                      edge_index_E, lp["E"], csr_e)
        bias = lp["occ"]["b"] + lp["N"]["b"] + lp["E"]["b"]
        cell = jax.nn.relu((s_occ + s_n + s_e)[:_N_CELL] + bias)
        piece = jax.nn.relu(piece)

    sums = jax.ops.segment_sum(cell, cell_batch, num_segments=_NG)
    cnt = jax.ops.segment_sum(
        jnp.ones((cell.shape[0],), cell.dtype), cell_batch, num_segments=_NG
    )
    graph_emb = sums / jnp.maximum(cnt, 1.0)[:, None]
    h = jax.nn.relu(_matmul(graph_emb, params["fc1_W"], params["fc1_b"], block_rows=64))
    policy_logits = _matmul(h, params["pol_W"], params["pol_b"], block_rows=64)
    value = jnp.tanh(_matmul(h, params["val_W"], params["val_b"], block_rows=64))
    return policy_logits, value
